# Initial kernel scaffold; baseline (speedup 1.0000x reference)
import functools

import jax
import jax.numpy as jnp
from jax import lax
from jax.experimental import pallas as pl
from jax.experimental.pallas import tpu as pltpu
from jax.experimental.pallas import tpu_sc as plsc

NC, NS = 2, 16
NW = NC * NS
CHUNK = 128
DEGW = 16


def _sc_mesh():
    return plsc.VectorSubcoreMesh(core_axis_name="c", subcore_axis_name="s")


def _make_deg_kernel(n_pad, nchunk):
    rows_per_tile = n_pad // NS

    @functools.partial(
        pl.kernel,
        out_type=jax.ShapeDtypeStruct((NC, n_pad, DEGW), jnp.float32),
        mesh=_sc_mesh(),
        scratch_types=[
            pltpu.VMEM((nchunk, CHUNK), jnp.int32),
            pltpu.VMEM((CHUNK, DEGW), jnp.float32),
            pltpu.VMEM((rows_per_tile, DEGW), jnp.float32),
            pltpu.VMEM_SHARED((n_pad, DEGW), jnp.float32),
        ],
    )
    def deg_kernel(dst_hbm, deg_out, dst_v, ones_v, bounce_v, deg_sp):
        cid = lax.axis_index("c")
        sid = lax.axis_index("s")
        wid = sid * NC + cid

        ones16 = jnp.ones((16,), jnp.float32)
        zeros16 = jnp.zeros((16,), jnp.float32)

        def fill_ones(i, _):
            ones_v[i] = ones16
            return 0

        def fill_zeros(i, _):
            bounce_v[i] = zeros16
            return 0

        lax.fori_loop(0, CHUNK, fill_ones, 0)
        lax.fori_loop(0, rows_per_tile, fill_zeros, 0)

        pltpu.sync_copy(bounce_v, deg_sp.at[pl.ds(sid * rows_per_tile, rows_per_tile)])
        pltpu.sync_copy(dst_hbm.at[wid], dst_v)
        plsc.subcore_barrier()

        def body(j, _):
            pltpu.sync_copy(ones_v, deg_sp.at[dst_v.at[j]], add=True)
            return 0

        lax.fori_loop(0, nchunk, body, 0)
        plsc.subcore_barrier()

        sl = pl.ds(sid * rows_per_tile, rows_per_tile)
        pltpu.sync_copy(deg_sp.at[sl], bounce_v)
        pltpu.sync_copy(bounce_v, deg_out.at[cid].at[sl])

    return deg_kernel


def _make_agg_kernel(n_pad, nchunk, dc):
    rows_per_tile = n_pad // NS

    @functools.partial(
        pl.kernel,
        out_type=jax.ShapeDtypeStruct((NC, n_pad, dc), jnp.float32),
        mesh=_sc_mesh(),
        scratch_types=[
            pltpu.VMEM((nchunk, CHUNK), jnp.int32),
            pltpu.VMEM((nchunk, CHUNK), jnp.int32),
            pltpu.VMEM((CHUNK, dc), jnp.float32),
            pltpu.VMEM((rows_per_tile, dc), jnp.float32),
            pltpu.VMEM_SHARED((n_pad, dc), jnp.float32),
            pltpu.SemaphoreType.DMA,
        ],
    )
    def agg_kernel(src_hbm, dst_hbm, g_hbm, s_out, src_v, dst_v, rows_v,
                   bounce_v, s_sp, sem):
        cid = lax.axis_index("c")
        sid = lax.axis_index("s")
        wid = sid * NC + cid

        sl = pl.ds(sid * rows_per_tile, rows_per_tile)
        pltpu.sync_copy(g_hbm.at[sl], bounce_v)
        pltpu.sync_copy(bounce_v, s_sp.at[sl])
        pltpu.sync_copy(src_hbm.at[wid], src_v)
        pltpu.sync_copy(dst_hbm.at[wid], dst_v)
        plsc.subcore_barrier()

        def body(j, _):
            pltpu.async_copy(g_hbm.at[src_v.at[j]], rows_v, sem).wait()
            pltpu.sync_copy(rows_v, s_sp.at[dst_v.at[j]], add=True)
            return 0

        lax.fori_loop(0, nchunk, body, 0)
        plsc.subcore_barrier()

        pltpu.sync_copy(s_sp.at[sl], bounce_v)
        pltpu.sync_copy(bounce_v, s_out.at[cid].at[sl])

    return agg_kernel


def _matmul_stage(x_pad, wcat, deg, n_pad, dc, block):
    nblk = n_pad // block

    def body(x_ref, w_ref, deg_ref, g_ref):
        degsum = deg_ref[0, :, 0] + deg_ref[1, :, 0] + 1.0
        dinv = lax.rsqrt(degsum)
        h = jnp.dot(x_ref[...], w_ref[...], preferred_element_type=jnp.float32)
        g_ref[...] = h * dinv[:, None]

    return pl.pallas_call(
        body,
        grid=(nblk,),
        in_specs=[
            pl.BlockSpec((block, x_pad.shape[1]), lambda i: (i, 0)),
            pl.BlockSpec((wcat.shape[0], dc), lambda i: (0, 0)),
            pl.BlockSpec((NC, block, DEGW), lambda i: (0, i, 0)),
        ],
        out_specs=pl.BlockSpec((block, dc), lambda i: (i, 0)),
        out_shape=jax.ShapeDtypeStruct((n_pad, dc), jnp.float32),
    )(x_pad, wcat, deg)


def _finalize_stage(s_parts, g, deg, bcat, n_pad, dc, block):
    nblk = n_pad // block

    def body(s_ref, g_ref, deg_ref, b_ref, out_ref):
        degsum = deg_ref[0, :, 0] + deg_ref[1, :, 0] + 1.0
        dinv = lax.rsqrt(degsum)
        tot = s_ref[0] + s_ref[1] - g_ref[...]
        out_ref[...] = tot * dinv[:, None] + b_ref[...]

    return pl.pallas_call(
        body,
        grid=(nblk,),
        in_specs=[
            pl.BlockSpec((NC, block, dc), lambda i: (0, i, 0)),
            pl.BlockSpec((block, dc), lambda i: (i, 0)),
            pl.BlockSpec((NC, block, DEGW), lambda i: (0, i, 0)),
            pl.BlockSpec((1, dc), lambda i: (0, 0)),
        ],
        out_specs=pl.BlockSpec((block, dc), lambda i: (i, 0)),
        out_shape=jax.ShapeDtypeStruct((n_pad, dc), jnp.float32),
    )(s_parts, g, deg, bcat)


@jax.jit
def kernel(x, W_mu, b_mu, W_logstd, b_logstd, edge_index):
    n, din = x.shape
    dout = W_mu.shape[1]
    dc = 2 * dout
    e = edge_index.shape[1]

    block = 1024
    n_pad = ((n + 1 + block - 1) // block) * block
    e_pad = ((e + NW * CHUNK - 1) // (NW * CHUNK)) * (NW * CHUNK)
    nchunk = e_pad // (NW * CHUNK)

    src = edge_index[0].astype(jnp.int32)
    dst = edge_index[1].astype(jnp.int32)
    src = jnp.concatenate([src, jnp.zeros((e_pad - e,), jnp.int32)])
    dst = jnp.concatenate([dst, jnp.full((e_pad - e,), n_pad - 1, jnp.int32)])
    src = src.reshape(NW, nchunk, CHUNK)
    dst = dst.reshape(NW, nchunk, CHUNK)

    x_pad = jnp.pad(x, ((0, n_pad - n), (0, 0)))
    wcat = jnp.concatenate([W_mu, W_logstd], axis=1)
    bcat = jnp.concatenate([b_mu, b_logstd]).reshape(1, dc)

    deg = _make_deg_kernel(n_pad, nchunk)(dst)
    g = _matmul_stage(x_pad, wcat, deg, n_pad, dc, block)
    s_parts = _make_agg_kernel(n_pad, nchunk, dc)(src, dst, g)
    out = _finalize_stage(s_parts, g, deg, bcat, n_pad, dc, block)

    return (out[:n, :dout], out[:n, dout:])


# speedup vs baseline: 26.1818x; 26.1818x over previous
import functools

import jax
import jax.numpy as jnp
from jax import lax
from jax.experimental import pallas as pl
from jax.experimental.pallas import tpu as pltpu
from jax.experimental.pallas import tpu_sc as plsc

NC, NS = 2, 16
NW = NC * NS
CHUNK = 128
DEGW = 16


def _sc_mesh():
    return plsc.VectorSubcoreMesh(
        core_axis_name="c", subcore_axis_name="s", num_cores=NC, num_subcores=NS
    )


def _make_deg_kernel(n_pad, nchunk):
    rows_per_tile = n_pad // NS

    @functools.partial(
        pl.kernel,
        out_type=jax.ShapeDtypeStruct((NC * n_pad, DEGW), jnp.float32),
        mesh=_sc_mesh(),
        scratch_types=[
            pltpu.VMEM((nchunk, CHUNK), jnp.int32),
            pltpu.VMEM((CHUNK, DEGW), jnp.float32),
            pltpu.VMEM((rows_per_tile, DEGW), jnp.float32),
            pltpu.VMEM_SHARED((n_pad, DEGW), jnp.float32),
        ],
        compiler_params=pltpu.CompilerParams(use_tc_tiling_on_sc=False),
    )
    def deg_kernel(dst_hbm, deg_out, dst_v, ones_v, bounce_v, deg_sp):
        cid = lax.axis_index("c")
        sid = lax.axis_index("s")
        wid = sid * NC + cid

        ones16 = jnp.ones((16,), jnp.float32)
        zeros16 = jnp.zeros((16,), jnp.float32)

        def fill_ones(i, _):
            ones_v[i] = ones16
            return 0

        def fill_zeros(i, _):
            bounce_v[i] = zeros16
            return 0

        lax.fori_loop(0, CHUNK, fill_ones, 0)
        lax.fori_loop(0, rows_per_tile, fill_zeros, 0)

        pltpu.sync_copy(bounce_v, deg_sp.at[pl.ds(sid * rows_per_tile, rows_per_tile)])
        pltpu.sync_copy(dst_hbm.at[wid], dst_v)
        plsc.subcore_barrier()

        def body(j, _):
            pltpu.sync_copy(ones_v, deg_sp.at[dst_v.at[j]], add=True)
            return 0

        lax.fori_loop(0, nchunk, body, 0)
        plsc.subcore_barrier()

        sl = pl.ds(sid * rows_per_tile, rows_per_tile)
        pltpu.sync_copy(deg_sp.at[sl], bounce_v)
        pltpu.sync_copy(
            bounce_v, deg_out.at[pl.ds(cid * n_pad + sid * rows_per_tile, rows_per_tile)]
        )

    return deg_kernel


def _make_agg_kernel(n_pad, nchunk, dc):
    rows_per_tile = n_pad // NS

    @functools.partial(
        pl.kernel,
        out_type=jax.ShapeDtypeStruct((NC * n_pad, dc), jnp.float32),
        mesh=_sc_mesh(),
        scratch_types=[
            pltpu.VMEM((nchunk, CHUNK), jnp.int32),
            pltpu.VMEM((nchunk, CHUNK), jnp.int32),
            pltpu.VMEM((CHUNK, dc), jnp.float32),
            pltpu.VMEM((rows_per_tile, dc), jnp.float32),
            pltpu.VMEM_SHARED((n_pad, dc), jnp.float32),
            pltpu.SemaphoreType.DMA,
        ],
        compiler_params=pltpu.CompilerParams(use_tc_tiling_on_sc=False),
    )
    def agg_kernel(src_hbm, dst_hbm, g_hbm, s_out, src_v, dst_v, rows_v,
                   bounce_v, s_sp, sem):
        cid = lax.axis_index("c")
        sid = lax.axis_index("s")
        wid = sid * NC + cid

        sl = pl.ds(sid * rows_per_tile, rows_per_tile)
        pltpu.sync_copy(g_hbm.at[sl], bounce_v)
        pltpu.sync_copy(bounce_v, s_sp.at[sl])
        pltpu.sync_copy(src_hbm.at[wid], src_v)
        pltpu.sync_copy(dst_hbm.at[wid], dst_v)
        plsc.subcore_barrier()

        def body(j, _):
            pltpu.async_copy(g_hbm.at[src_v.at[j]], rows_v, sem).wait()
            pltpu.sync_copy(rows_v, s_sp.at[dst_v.at[j]], add=True)
            return 0

        lax.fori_loop(0, nchunk, body, 0)
        plsc.subcore_barrier()

        pltpu.sync_copy(s_sp.at[sl], bounce_v)
        pltpu.sync_copy(
            bounce_v, s_out.at[pl.ds(cid * n_pad + sid * rows_per_tile, rows_per_tile)]
        )

    return agg_kernel


def _matmul_stage(x_pad, wcat, deg, n_pad, dc, block):
    nblk = n_pad // block

    def body(x_ref, w_ref, deg_ref, g_ref):
        degsum = deg_ref[0, :, 0] + deg_ref[1, :, 0] + 1.0
        dinv = lax.rsqrt(degsum)
        h = jnp.dot(x_ref[...], w_ref[...], preferred_element_type=jnp.float32)
        g_ref[...] = h * dinv[:, None]

    return pl.pallas_call(
        body,
        grid=(nblk,),
        in_specs=[
            pl.BlockSpec((block, x_pad.shape[1]), lambda i: (i, 0)),
            pl.BlockSpec((wcat.shape[0], dc), lambda i: (0, 0)),
            pl.BlockSpec((NC, block, DEGW), lambda i: (0, i, 0)),
        ],
        out_specs=pl.BlockSpec((block, dc), lambda i: (i, 0)),
        out_shape=jax.ShapeDtypeStruct((n_pad, dc), jnp.float32),
    )(x_pad, wcat, deg)


def _finalize_stage(s_parts, g, deg, bcat, n_pad, dc, block):
    nblk = n_pad // block

    def body(s_ref, g_ref, deg_ref, b_ref, out_ref):
        degsum = deg_ref[0, :, 0] + deg_ref[1, :, 0] + 1.0
        dinv = lax.rsqrt(degsum)
        tot = s_ref[0] + s_ref[1] - g_ref[...]
        out_ref[...] = tot * dinv[:, None] + b_ref[...]

    return pl.pallas_call(
        body,
        grid=(nblk,),
        in_specs=[
            pl.BlockSpec((NC, block, dc), lambda i: (0, i, 0)),
            pl.BlockSpec((block, dc), lambda i: (i, 0)),
            pl.BlockSpec((NC, block, DEGW), lambda i: (0, i, 0)),
            pl.BlockSpec((1, dc), lambda i: (0, 0)),
        ],
        out_specs=pl.BlockSpec((block, dc), lambda i: (i, 0)),
        out_shape=jax.ShapeDtypeStruct((n_pad, dc), jnp.float32),
    )(s_parts, g, deg, bcat)


@jax.jit
def kernel(x, W_mu, b_mu, W_logstd, b_logstd, edge_index):
    n, din = x.shape
    dout = W_mu.shape[1]
    dc = 2 * dout
    e = edge_index.shape[1]

    block = 1024
    n_pad = ((n + 1 + block - 1) // block) * block
    e_pad = ((e + NW * CHUNK - 1) // (NW * CHUNK)) * (NW * CHUNK)
    nchunk = e_pad // (NW * CHUNK)

    src = edge_index[0].astype(jnp.int32)
    dst = edge_index[1].astype(jnp.int32)
    src = jnp.concatenate([src, jnp.zeros((e_pad - e,), jnp.int32)])
    dst = jnp.concatenate([dst, jnp.full((e_pad - e,), n_pad - 1, jnp.int32)])
    src = src.reshape(NW, nchunk, CHUNK)
    dst = dst.reshape(NW, nchunk, CHUNK)

    x_pad = jnp.pad(x, ((0, n_pad - n), (0, 0)))
    wcat = jnp.concatenate([W_mu, W_logstd], axis=1)
    bcat = jnp.concatenate([b_mu, b_logstd]).reshape(1, dc)

    deg = _make_deg_kernel(n_pad, nchunk)(dst).reshape(NC, n_pad, DEGW)
    g = _matmul_stage(x_pad, wcat, deg, n_pad, dc, block)
    s_parts = _make_agg_kernel(n_pad, nchunk, dc)(src, dst, g).reshape(NC, n_pad, dc)
    out = _finalize_stage(s_parts, g, deg, bcat, n_pad, dc, block)

    return (out[:n, :dout], out[:n, dout:])


# speedup vs baseline: 30.2786x; 1.1565x over previous
import functools

import jax
import jax.numpy as jnp
from jax import lax
from jax.experimental import pallas as pl
from jax.experimental.pallas import tpu as pltpu
from jax.experimental.pallas import tpu_sc as plsc

NC, NS = 2, 16
NW = NC * NS
CHUNK = 128
DEGW = 16
DEPTH = 4
SP_PAD = 10240


def _sc_mesh():
    return plsc.VectorSubcoreMesh(
        core_axis_name="c", subcore_axis_name="s", num_cores=NC, num_subcores=NS
    )


def _make_deg_kernel(n, nchunk):
    out_rows = n // NS
    zrows = SP_PAD // NS

    @functools.partial(
        pl.kernel,
        out_type=jax.ShapeDtypeStruct((NC * n, DEGW), jnp.float32),
        mesh=_sc_mesh(),
        scratch_types=[
            pltpu.VMEM((nchunk, CHUNK), jnp.int32),
            pltpu.VMEM((CHUNK, DEGW), jnp.float32),
            pltpu.VMEM((out_rows, DEGW), jnp.float32),
            pltpu.VMEM_SHARED((SP_PAD, DEGW), jnp.float32),
        ]
        + [pltpu.SemaphoreType.DMA] * DEPTH,
        compiler_params=pltpu.CompilerParams(use_tc_tiling_on_sc=False),
    )
    def deg_kernel(dst_hbm, deg_out, dst_v, ones_v, bounce_v, deg_sp, *sems):
        cid = lax.axis_index("c")
        sid = lax.axis_index("s")
        wid = sid * NC + cid

        ones16 = jnp.ones((16,), jnp.float32)
        zeros16 = jnp.zeros((16,), jnp.float32)

        def fill(i, _):
            ones_v[i] = ones16
            bounce_v[i] = zeros16
            return 0

        lax.fori_loop(0, CHUNK, fill, 0)

        zsrc = bounce_v.at[pl.ds(0, CHUNK)]
        for r in range(zrows // CHUNK):
            pltpu.sync_copy(
                zsrc, deg_sp.at[pl.ds(sid * zrows + r * CHUNK, CHUNK)]
            )
        pltpu.sync_copy(dst_hbm.at[wid], dst_v)
        plsc.subcore_barrier()

        def body(t, _):
            for k in range(DEPTH):
                j = DEPTH * t + k
                pltpu.async_copy(
                    ones_v, deg_sp.at[dst_v.at[j]], sems[k], add=True
                )
            for k in range(DEPTH):
                j = DEPTH * t + k
                pltpu.make_async_copy(
                    ones_v, deg_sp.at[dst_v.at[j]], sems[k]
                ).wait()
            return 0

        lax.fori_loop(0, nchunk // DEPTH, body, 0)
        plsc.subcore_barrier()

        sl = pl.ds(sid * out_rows, out_rows)
        pltpu.sync_copy(deg_sp.at[sl], bounce_v)
        pltpu.sync_copy(
            bounce_v, deg_out.at[pl.ds(cid * n + sid * out_rows, out_rows)]
        )

    return deg_kernel


def _make_agg_kernel(n, nchunk, dc):
    out_rows = n // NS
    zrows = SP_PAD // NS

    @functools.partial(
        pl.kernel,
        out_type=jax.ShapeDtypeStruct((NC * n, dc), jnp.float32),
        mesh=_sc_mesh(),
        scratch_types=[
            pltpu.VMEM((nchunk, CHUNK), jnp.int32),
            pltpu.VMEM((nchunk, CHUNK), jnp.int32),
            pltpu.VMEM((DEPTH, CHUNK, dc), jnp.float32),
            pltpu.VMEM((out_rows, dc), jnp.float32),
            pltpu.VMEM_SHARED((SP_PAD, dc), jnp.float32),
        ]
        + [pltpu.SemaphoreType.DMA] * DEPTH,
        compiler_params=pltpu.CompilerParams(use_tc_tiling_on_sc=False),
    )
    def agg_kernel(src_hbm, dst_hbm, g_hbm, s_out, src_v, dst_v, bufs,
                   bounce_v, s_sp, *sems):
        cid = lax.axis_index("c")
        sid = lax.axis_index("s")
        wid = sid * NC + cid

        zeros16 = jnp.zeros((16,), jnp.float32)

        def fill(i, _):
            for c0 in range(0, dc, 16):
                bounce_v[i, pl.ds(c0, 16)] = zeros16
            return 0

        lax.fori_loop(0, CHUNK, fill, 0)

        zsrc = bounce_v.at[pl.ds(0, CHUNK)]
        for r in range(zrows // CHUNK):
            pltpu.sync_copy(zsrc, s_sp.at[pl.ds(sid * zrows + r * CHUNK, CHUNK)])
        pltpu.sync_copy(src_hbm.at[wid], src_v)
        pltpu.sync_copy(dst_hbm.at[wid], dst_v)
        plsc.subcore_barrier()

        for k in range(DEPTH):
            pltpu.async_copy(g_hbm.at[src_v.at[k]], bufs.at[k], sems[k])

        def body(t, _):
            for k in range(DEPTH):
                j = DEPTH * t + k
                pltpu.make_async_copy(
                    g_hbm.at[src_v.at[j]], bufs.at[k], sems[k]
                ).wait()
                pltpu.sync_copy(bufs.at[k], s_sp.at[dst_v.at[j]], add=True)

                @pl.when(j + DEPTH < nchunk)
                def _():
                    pltpu.async_copy(
                        g_hbm.at[src_v.at[j + DEPTH]], bufs.at[k], sems[k]
                    )

            return 0

        lax.fori_loop(0, nchunk // DEPTH, body, 0)
        plsc.subcore_barrier()

        sl = pl.ds(sid * out_rows, out_rows)
        pltpu.sync_copy(s_sp.at[sl], bounce_v)
        pltpu.sync_copy(
            bounce_v, s_out.at[pl.ds(cid * n + sid * out_rows, out_rows)]
        )

    return agg_kernel


def _matmul_stage(x, wcat, n, dc, block):
    def body(x_ref, w_ref, h_ref):
        h_ref[...] = jnp.dot(
            x_ref[...], w_ref[...], preferred_element_type=jnp.float32
        )

    return pl.pallas_call(
        body,
        grid=(n // block,),
        in_specs=[
            pl.BlockSpec((block, x.shape[1]), lambda i: (i, 0)),
            pl.BlockSpec((wcat.shape[0], dc), lambda i: (0, 0)),
        ],
        out_specs=pl.BlockSpec((block, dc), lambda i: (i, 0)),
        out_shape=jax.ShapeDtypeStruct((n, dc), jnp.float32),
    )(x, wcat)


def _scale_stage(h, deg, n, dc, block):
    nblk = n // block

    def body(h_ref, d0_ref, d1_ref, g_ref):
        degsum = d0_ref[:, 0] + d1_ref[:, 0] + 1.0
        g_ref[...] = h_ref[...] * lax.rsqrt(degsum)[:, None]

    return pl.pallas_call(
        body,
        grid=(nblk,),
        in_specs=[
            pl.BlockSpec((block, dc), lambda i: (i, 0)),
            pl.BlockSpec((block, DEGW), lambda i: (i, 0)),
            pl.BlockSpec((block, DEGW), lambda i, _n=nblk: (i + _n, 0)),
        ],
        out_specs=pl.BlockSpec((block, dc), lambda i: (i, 0)),
        out_shape=jax.ShapeDtypeStruct((n, dc), jnp.float32),
    )(h, deg, deg)


def _finalize_stage(s_parts, g, deg, b_mu, b_logstd, n, dc, dout, block):
    nblk = n // block

    def body(s0_ref, s1_ref, g_ref, d0_ref, d1_ref, bm_ref, bl_ref,
             mu_ref, lo_ref):
        degsum = d0_ref[:, 0] + d1_ref[:, 0] + 1.0
        dinv = lax.rsqrt(degsum)
        out = (s0_ref[...] + s1_ref[...] + g_ref[...]) * dinv[:, None]
        mu_ref[...] = out[:, :dout] + bm_ref[...]
        lo_ref[...] = out[:, dout:] + bl_ref[...]

    return pl.pallas_call(
        body,
        grid=(nblk,),
        in_specs=[
            pl.BlockSpec((block, dc), lambda i: (i, 0)),
            pl.BlockSpec((block, dc), lambda i, _n=nblk: (i + _n, 0)),
            pl.BlockSpec((block, dc), lambda i: (i, 0)),
            pl.BlockSpec((block, DEGW), lambda i: (i, 0)),
            pl.BlockSpec((block, DEGW), lambda i, _n=nblk: (i + _n, 0)),
            pl.BlockSpec((1, dout), lambda i: (0, 0)),
            pl.BlockSpec((1, dout), lambda i: (0, 0)),
        ],
        out_specs=[
            pl.BlockSpec((block, dout), lambda i: (i, 0)),
            pl.BlockSpec((block, dout), lambda i: (i, 0)),
        ],
        out_shape=[
            jax.ShapeDtypeStruct((n, dout), jnp.float32),
            jax.ShapeDtypeStruct((n, dout), jnp.float32),
        ],
    )(s_parts, s_parts, g, deg, deg, b_mu, b_logstd)


@jax.jit
def kernel(x, W_mu, b_mu, W_logstd, b_logstd, edge_index):
    n, din = x.shape
    dout = W_mu.shape[1]
    dc = 2 * dout
    e = edge_index.shape[1]
    block = 1000

    e_pad = ((e + NW * CHUNK - 1) // (NW * CHUNK)) * (NW * CHUNK)
    nchunk = e_pad // (NW * CHUNK)

    src = edge_index[0].astype(jnp.int32)
    dst = edge_index[1].astype(jnp.int32)
    src = jnp.concatenate([src, jnp.zeros((e_pad - e,), jnp.int32)])
    dst = jnp.concatenate([dst, jnp.full((e_pad - e,), SP_PAD - 1, jnp.int32)])
    src = src.reshape(NW, nchunk, CHUNK)
    dst = dst.reshape(NW, nchunk, CHUNK)

    wcat = jnp.concatenate([W_mu, W_logstd], axis=1)

    h = _matmul_stage(x, wcat, n, dc, block)
    deg = _make_deg_kernel(n, nchunk)(dst)
    g = _scale_stage(h, deg, n, dc, block)
    s_parts = _make_agg_kernel(n, nchunk, dc)(src, dst, g)
    return _finalize_stage(
        s_parts, g, deg, b_mu.reshape(1, dout), b_logstd.reshape(1, dout),
        n, dc, dout, block
    )


# speedup vs baseline: 32.3644x; 1.0689x over previous
import functools

import jax
import jax.numpy as jnp
from jax import lax
from jax.experimental import pallas as pl
from jax.experimental.pallas import tpu as pltpu
from jax.experimental.pallas import tpu_sc as plsc

NC, NS = 2, 16
NW = NC * NS
CHUNK = 128
DEGW = 8
DEPTH = 8
SP_PAD = 10240


def _sc_mesh():
    return plsc.VectorSubcoreMesh(
        core_axis_name="c", subcore_axis_name="s", num_cores=NC, num_subcores=NS
    )


def _make_deg_kernel(n, nchunk):
    out_rows = n // NS
    zrows = SP_PAD // NS

    @functools.partial(
        pl.kernel,
        out_type=jax.ShapeDtypeStruct((NC * n, DEGW), jnp.float32),
        mesh=_sc_mesh(),
        scratch_types=[
            pltpu.VMEM((nchunk, CHUNK), jnp.int32),
            pltpu.VMEM((CHUNK, DEGW), jnp.float32),
            pltpu.VMEM((out_rows, DEGW), jnp.float32),
            pltpu.VMEM_SHARED((SP_PAD, DEGW), jnp.float32),
        ]
        + [pltpu.SemaphoreType.DMA] * DEPTH,
        compiler_params=pltpu.CompilerParams(use_tc_tiling_on_sc=False),
    )
    def deg_kernel(dst_hbm, ones_hbm, deg_out, dst_v, ones_v, bounce_v,
                   deg_sp, *sems):
        cid = lax.axis_index("c")
        sid = lax.axis_index("s")
        wid = sid * NC + cid

        pltpu.sync_copy(ones_hbm.at[0], ones_v)
        pltpu.sync_copy(ones_hbm.at[1], bounce_v.at[pl.ds(0, CHUNK)])

        zsrc = bounce_v.at[pl.ds(0, CHUNK)]
        for r in range(zrows // CHUNK):
            pltpu.sync_copy(
                zsrc, deg_sp.at[pl.ds(sid * zrows + r * CHUNK, CHUNK)]
            )
        pltpu.sync_copy(dst_hbm.at[wid], dst_v)
        plsc.subcore_barrier()

        def body(t, _):
            for k in range(DEPTH):
                j = DEPTH * t + k
                pltpu.async_copy(
                    ones_v, deg_sp.at[dst_v.at[j]], sems[k], add=True
                )
            for k in range(DEPTH):
                j = DEPTH * t + k
                pltpu.make_async_copy(
                    ones_v, deg_sp.at[dst_v.at[j]], sems[k]
                ).wait()
            return 0

        lax.fori_loop(0, nchunk // DEPTH, body, 0)
        plsc.subcore_barrier()

        sl = pl.ds(sid * out_rows, out_rows)
        pltpu.sync_copy(deg_sp.at[sl], bounce_v)
        pltpu.sync_copy(
            bounce_v, deg_out.at[pl.ds(cid * n + sid * out_rows, out_rows)]
        )

    return deg_kernel


def _make_agg_kernel(n, nchunk, dc):
    out_rows = n // NS
    zrows = SP_PAD // NS

    @functools.partial(
        pl.kernel,
        out_type=jax.ShapeDtypeStruct((NC * n, dc), jnp.float32),
        mesh=_sc_mesh(),
        scratch_types=[
            pltpu.VMEM((nchunk, CHUNK), jnp.int32),
            pltpu.VMEM((nchunk, CHUNK), jnp.int32),
            pltpu.VMEM((DEPTH, CHUNK, dc), jnp.float32),
            pltpu.VMEM((out_rows, dc), jnp.float32),
            pltpu.VMEM_SHARED((SP_PAD, dc), jnp.float32),
        ]
        + [pltpu.SemaphoreType.DMA] * DEPTH,
        compiler_params=pltpu.CompilerParams(use_tc_tiling_on_sc=False),
    )
    def agg_kernel(src_hbm, dst_hbm, g_hbm, s_out, src_v, dst_v, bufs,
                   bounce_v, s_sp, *sems):
        cid = lax.axis_index("c")
        sid = lax.axis_index("s")
        wid = sid * NC + cid

        zeros16 = jnp.zeros((16,), jnp.float32)

        def fill(i, _):
            for c0 in range(0, dc, 16):
                bounce_v[i, pl.ds(c0, 16)] = zeros16
            return 0

        lax.fori_loop(0, CHUNK, fill, 0)

        zsrc = bounce_v.at[pl.ds(0, CHUNK)]
        for r in range(zrows // CHUNK):
            pltpu.sync_copy(zsrc, s_sp.at[pl.ds(sid * zrows + r * CHUNK, CHUNK)])
        pltpu.sync_copy(src_hbm.at[wid], src_v)
        pltpu.sync_copy(dst_hbm.at[wid], dst_v)
        plsc.subcore_barrier()

        for k in range(DEPTH):
            pltpu.async_copy(g_hbm.at[src_v.at[k]], bufs.at[k], sems[k])

        def body(t, _):
            for k in range(DEPTH):
                j = DEPTH * t + k
                pltpu.make_async_copy(
                    g_hbm.at[src_v.at[j]], bufs.at[k], sems[k]
                ).wait()
                pltpu.sync_copy(bufs.at[k], s_sp.at[dst_v.at[j]], add=True)

                @pl.when(j + DEPTH < nchunk)
                def _():
                    pltpu.async_copy(
                        g_hbm.at[src_v.at[j + DEPTH]], bufs.at[k], sems[k]
                    )

            return 0

        lax.fori_loop(0, nchunk // DEPTH, body, 0)
        plsc.subcore_barrier()

        sl = pl.ds(sid * out_rows, out_rows)
        pltpu.sync_copy(s_sp.at[sl], bounce_v)
        pltpu.sync_copy(
            bounce_v, s_out.at[pl.ds(cid * n + sid * out_rows, out_rows)]
        )

    return agg_kernel


def _matmul_stage(x, wcat, n, dc, block):
    def body(x_ref, w_ref, h_ref):
        h_ref[...] = jnp.dot(
            x_ref[...], w_ref[...], preferred_element_type=jnp.float32
        )

    return pl.pallas_call(
        body,
        grid=(n // block,),
        in_specs=[
            pl.BlockSpec((block, x.shape[1]), lambda i: (i, 0)),
            pl.BlockSpec((wcat.shape[0], dc), lambda i: (0, 0)),
        ],
        out_specs=pl.BlockSpec((block, dc), lambda i: (i, 0)),
        out_shape=jax.ShapeDtypeStruct((n, dc), jnp.float32),
    )(x, wcat)


def _scale_stage(h, deg, n, dc):
    def body(h_ref, d0_ref, d1_ref, g_ref):
        degsum = d0_ref[:, 0] + d1_ref[:, 0] + 1.0
        g_ref[...] = h_ref[...] * lax.rsqrt(degsum)[:, None]

    return pl.pallas_call(
        body,
        grid=(1,),
        in_specs=[
            pl.BlockSpec((n, dc), lambda i: (0, 0)),
            pl.BlockSpec((n, DEGW), lambda i: (0, 0)),
            pl.BlockSpec((n, DEGW), lambda i: (1, 0)),
        ],
        out_specs=pl.BlockSpec((n, dc), lambda i: (0, 0)),
        out_shape=jax.ShapeDtypeStruct((n, dc), jnp.float32),
    )(h, deg, deg)


def _finalize_stage(s_parts, g, deg, b_mu, b_logstd, n, dc, dout):
    def body(s0_ref, s1_ref, g_ref, d0_ref, d1_ref, bm_ref, bl_ref,
             mu_ref, lo_ref):
        degsum = d0_ref[:, 0] + d1_ref[:, 0] + 1.0
        dinv = lax.rsqrt(degsum)
        out = (s0_ref[...] + s1_ref[...] + g_ref[...]) * dinv[:, None]
        mu_ref[...] = out[:, :dout] + bm_ref[...]
        lo_ref[...] = out[:, dout:] + bl_ref[...]

    return pl.pallas_call(
        body,
        grid=(1,),
        in_specs=[
            pl.BlockSpec((n, dc), lambda i: (0, 0)),
            pl.BlockSpec((n, dc), lambda i: (1, 0)),
            pl.BlockSpec((n, dc), lambda i: (0, 0)),
            pl.BlockSpec((n, DEGW), lambda i: (0, 0)),
            pl.BlockSpec((n, DEGW), lambda i: (1, 0)),
            pl.BlockSpec((1, dout), lambda i: (0, 0)),
            pl.BlockSpec((1, dout), lambda i: (0, 0)),
        ],
        out_specs=[
            pl.BlockSpec((n, dout), lambda i: (0, 0)),
            pl.BlockSpec((n, dout), lambda i: (0, 0)),
        ],
        out_shape=[
            jax.ShapeDtypeStruct((n, dout), jnp.float32),
            jax.ShapeDtypeStruct((n, dout), jnp.float32),
        ],
    )(s_parts, s_parts, g, deg, deg, b_mu, b_logstd)


@jax.jit
def kernel(x, W_mu, b_mu, W_logstd, b_logstd, edge_index):
    n, din = x.shape
    dout = W_mu.shape[1]
    dc = 2 * dout
    e = edge_index.shape[1]
    block = 1000

    e_pad = ((e + NW * CHUNK - 1) // (NW * CHUNK)) * (NW * CHUNK)
    nchunk = e_pad // (NW * CHUNK)

    src = edge_index[0].astype(jnp.int32)
    dst = edge_index[1].astype(jnp.int32)
    src = jnp.concatenate([src, jnp.zeros((e_pad - e,), jnp.int32)])
    dst = jnp.concatenate([dst, jnp.full((e_pad - e,), SP_PAD - 1, jnp.int32)])
    src = src.reshape(NW, nchunk, CHUNK)
    dst = dst.reshape(NW, nchunk, CHUNK)

    wcat = jnp.concatenate([W_mu, W_logstd], axis=1)

    oz = jnp.stack(
        [jnp.ones((CHUNK, DEGW), jnp.float32), jnp.zeros((CHUNK, DEGW), jnp.float32)]
    )
    h = _matmul_stage(x, wcat, n, dc, block)
    deg = _make_deg_kernel(n, nchunk)(dst, oz)
    g = _scale_stage(h, deg, n, dc)
    s_parts = _make_agg_kernel(n, nchunk, dc)(src, dst, g)
    return _finalize_stage(
        s_parts, g, deg, b_mu.reshape(1, dout), b_logstd.reshape(1, dout),
        n, dc, dout
    )


# speedup vs baseline: 39.9576x; 1.2346x over previous
import functools

import jax
import jax.numpy as jnp
from jax import lax
from jax.experimental import pallas as pl
from jax.experimental.pallas import tpu as pltpu
from jax.experimental.pallas import tpu_sc as plsc

NC, NS = 2, 16
NW = NC * NS
CHUNK = 128
DEGW = 8
DEPTH = 8
SP_PAD = 10240


def _sc_mesh():
    return plsc.VectorSubcoreMesh(
        core_axis_name="c", subcore_axis_name="s", num_cores=NC, num_subcores=NS
    )


def _make_deg_kernel(n, nchunk):
    out_rows = n // NS
    zrows = SP_PAD // NS

    @functools.partial(
        pl.kernel,
        out_type=jax.ShapeDtypeStruct((NC * n, DEGW), jnp.float32),
        mesh=_sc_mesh(),
        scratch_types=[
            pltpu.VMEM((nchunk, CHUNK), jnp.int32),
            pltpu.VMEM((CHUNK, DEGW), jnp.float32),
            pltpu.VMEM((out_rows, DEGW), jnp.float32),
            pltpu.VMEM_SHARED((SP_PAD, DEGW), jnp.float32),
        ]
        + [pltpu.SemaphoreType.DMA] * DEPTH,
        compiler_params=pltpu.CompilerParams(use_tc_tiling_on_sc=False),
    )
    def deg_kernel(dst_hbm, ones_hbm, deg_out, dst_v, ones_v, bounce_v,
                   deg_sp, *sems):
        cid = lax.axis_index("c")
        sid = lax.axis_index("s")
        wid = sid * NC + cid

        pltpu.sync_copy(ones_hbm.at[0], ones_v)
        pltpu.sync_copy(ones_hbm.at[1], bounce_v.at[pl.ds(0, CHUNK)])

        zsrc = bounce_v.at[pl.ds(0, CHUNK)]
        for r in range(zrows // CHUNK):
            pltpu.sync_copy(
                zsrc, deg_sp.at[pl.ds(sid * zrows + r * CHUNK, CHUNK)]
            )
        pltpu.sync_copy(dst_hbm.at[wid], dst_v)
        plsc.subcore_barrier()

        def body(t, _):
            for k in range(DEPTH):
                j = DEPTH * t + k
                pltpu.async_copy(
                    ones_v, deg_sp.at[dst_v.at[j]], sems[k], add=True
                )
            for k in range(DEPTH):
                j = DEPTH * t + k
                pltpu.make_async_copy(
                    ones_v, deg_sp.at[dst_v.at[j]], sems[k]
                ).wait()
            return 0

        lax.fori_loop(0, nchunk // DEPTH, body, 0)
        plsc.subcore_barrier()

        sl = pl.ds(sid * out_rows, out_rows)
        pltpu.sync_copy(deg_sp.at[sl], bounce_v)
        pltpu.sync_copy(
            bounce_v, deg_out.at[pl.ds(cid * n + sid * out_rows, out_rows)]
        )

    return deg_kernel


def _make_agg_kernel(n, nchunk, dc):
    out_rows = n // NS
    zrows = SP_PAD // NS

    @functools.partial(
        pl.kernel,
        out_type=jax.ShapeDtypeStruct((NC * n, dc), jnp.float32),
        mesh=_sc_mesh(),
        scratch_types=[
            pltpu.VMEM((nchunk, CHUNK), jnp.int32),
            pltpu.VMEM((nchunk, CHUNK), jnp.int32),
            pltpu.VMEM((DEPTH, CHUNK, dc), jnp.float32),
            pltpu.VMEM((out_rows, dc), jnp.float32),
            pltpu.VMEM_SHARED((SP_PAD, dc), jnp.float32),
            pltpu.VMEM_SHARED((n, dc), jnp.float32),
        ]
        + [pltpu.SemaphoreType.DMA] * DEPTH,
        compiler_params=pltpu.CompilerParams(use_tc_tiling_on_sc=False),
    )
    def agg_kernel(src_hbm, dst_hbm, g_hbm, s_out, src_v, dst_v, bufs,
                   bounce_v, s_sp, g_sp, *sems):
        cid = lax.axis_index("c")
        sid = lax.axis_index("s")
        wid = sid * NC + cid

        gsl = pl.ds(sid * out_rows, out_rows)
        pltpu.sync_copy(g_hbm.at[gsl], bounce_v)
        pltpu.sync_copy(bounce_v, g_sp.at[gsl])

        zeros16 = jnp.zeros((16,), jnp.float32)

        def fill(i, _):
            for c0 in range(0, dc, 16):
                bounce_v[i, pl.ds(c0, 16)] = zeros16
            return 0

        lax.fori_loop(0, CHUNK, fill, 0)

        zsrc = bounce_v.at[pl.ds(0, CHUNK)]
        for r in range(zrows // CHUNK):
            pltpu.sync_copy(zsrc, s_sp.at[pl.ds(sid * zrows + r * CHUNK, CHUNK)])
        pltpu.sync_copy(src_hbm.at[wid], src_v)
        pltpu.sync_copy(dst_hbm.at[wid], dst_v)
        plsc.subcore_barrier()

        for k in range(DEPTH):
            pltpu.async_copy(g_sp.at[src_v.at[k]], bufs.at[k], sems[k])

        def body(t, _):
            for k in range(DEPTH):
                j = DEPTH * t + k
                pltpu.make_async_copy(
                    g_sp.at[src_v.at[j]], bufs.at[k], sems[k]
                ).wait()
                pltpu.sync_copy(bufs.at[k], s_sp.at[dst_v.at[j]], add=True)

                @pl.when(j + DEPTH < nchunk)
                def _():
                    pltpu.async_copy(
                        g_sp.at[src_v.at[j + DEPTH]], bufs.at[k], sems[k]
                    )

            return 0

        lax.fori_loop(0, nchunk // DEPTH, body, 0)
        plsc.subcore_barrier()

        sl = pl.ds(sid * out_rows, out_rows)
        pltpu.sync_copy(s_sp.at[sl], bounce_v)
        pltpu.sync_copy(
            bounce_v, s_out.at[pl.ds(cid * n + sid * out_rows, out_rows)]
        )

    return agg_kernel


def _matmul_stage(x, wcat, n, dc, block):
    def body(x_ref, w_ref, h_ref):
        h_ref[...] = jnp.dot(
            x_ref[...], w_ref[...], preferred_element_type=jnp.float32
        )

    return pl.pallas_call(
        body,
        grid=(n // block,),
        in_specs=[
            pl.BlockSpec((block, x.shape[1]), lambda i: (i, 0)),
            pl.BlockSpec((wcat.shape[0], dc), lambda i: (0, 0)),
        ],
        out_specs=pl.BlockSpec((block, dc), lambda i: (i, 0)),
        out_shape=jax.ShapeDtypeStruct((n, dc), jnp.float32),
    )(x, wcat)


def _scale_stage(h, deg, n, dc):
    def body(h_ref, d0_ref, d1_ref, g_ref):
        degsum = d0_ref[:, 0] + d1_ref[:, 0] + 1.0
        g_ref[...] = h_ref[...] * lax.rsqrt(degsum)[:, None]

    return pl.pallas_call(
        body,
        grid=(1,),
        in_specs=[
            pl.BlockSpec((n, dc), lambda i: (0, 0)),
            pl.BlockSpec((n, DEGW), lambda i: (0, 0)),
            pl.BlockSpec((n, DEGW), lambda i: (1, 0)),
        ],
        out_specs=pl.BlockSpec((n, dc), lambda i: (0, 0)),
        out_shape=jax.ShapeDtypeStruct((n, dc), jnp.float32),
    )(h, deg, deg)


def _finalize_stage(s_parts, g, deg, b_mu, b_logstd, n, dc, dout):
    def body(s0_ref, s1_ref, g_ref, d0_ref, d1_ref, bm_ref, bl_ref,
             mu_ref, lo_ref):
        degsum = d0_ref[:, 0] + d1_ref[:, 0] + 1.0
        dinv = lax.rsqrt(degsum)
        out = (s0_ref[...] + s1_ref[...] + g_ref[...]) * dinv[:, None]
        mu_ref[...] = out[:, :dout] + bm_ref[...]
        lo_ref[...] = out[:, dout:] + bl_ref[...]

    return pl.pallas_call(
        body,
        grid=(1,),
        in_specs=[
            pl.BlockSpec((n, dc), lambda i: (0, 0)),
            pl.BlockSpec((n, dc), lambda i: (1, 0)),
            pl.BlockSpec((n, dc), lambda i: (0, 0)),
            pl.BlockSpec((n, DEGW), lambda i: (0, 0)),
            pl.BlockSpec((n, DEGW), lambda i: (1, 0)),
            pl.BlockSpec((1, dout), lambda i: (0, 0)),
            pl.BlockSpec((1, dout), lambda i: (0, 0)),
        ],
        out_specs=[
            pl.BlockSpec((n, dout), lambda i: (0, 0)),
            pl.BlockSpec((n, dout), lambda i: (0, 0)),
        ],
        out_shape=[
            jax.ShapeDtypeStruct((n, dout), jnp.float32),
            jax.ShapeDtypeStruct((n, dout), jnp.float32),
        ],
    )(s_parts, s_parts, g, deg, deg, b_mu, b_logstd)


@jax.jit
def kernel(x, W_mu, b_mu, W_logstd, b_logstd, edge_index):
    n, din = x.shape
    dout = W_mu.shape[1]
    dc = 2 * dout
    e = edge_index.shape[1]
    block = 1000

    e_pad = ((e + NW * CHUNK - 1) // (NW * CHUNK)) * (NW * CHUNK)
    nchunk = e_pad // (NW * CHUNK)

    src = edge_index[0].astype(jnp.int32)
    dst = edge_index[1].astype(jnp.int32)
    src = jnp.concatenate([src, jnp.zeros((e_pad - e,), jnp.int32)])
    dst = jnp.concatenate([dst, jnp.full((e_pad - e,), SP_PAD - 1, jnp.int32)])
    src = src.reshape(NW, nchunk, CHUNK)
    dst = dst.reshape(NW, nchunk, CHUNK)

    wcat = jnp.concatenate([W_mu, W_logstd], axis=1)

    oz = jnp.stack(
        [jnp.ones((CHUNK, DEGW), jnp.float32), jnp.zeros((CHUNK, DEGW), jnp.float32)]
    )
    h = _matmul_stage(x, wcat, n, dc, block)
    deg = _make_deg_kernel(n, nchunk)(dst, oz)
    g = _scale_stage(h, deg, n, dc)
    s_parts = _make_agg_kernel(n, nchunk, dc)(src, dst, g)
    return _finalize_stage(
        s_parts, g, deg, b_mu.reshape(1, dout), b_logstd.reshape(1, dout),
        n, dc, dout
    )
